# baseline (speedup 1.0000x reference)
import functools

import jax
import jax.numpy as jnp
from jax import lax
from jax.experimental import pallas as pl
from jax.experimental.pallas import tpu as pltpu
from jax.experimental.pallas import tpu_sc as plsc

H = 128
HH = H // 2
N = 10000
NC = 2
NT = 16
SLAB = 624
GRP = 128
SUB = 256
BLK = 12
NBLK = 7
EPT = SUB * BLK * NBLK
E_PAD = EPT * NT
IDXR = 8


def _mlp_body(x_ref, mw1_ref, mb1_ref, mw2_ref, mb2_ref,
              sw1_ref, sb1_ref, sw2_ref, sb2_ref, m2_ref, s2_ref):
    xb = x_ref[...]
    h = jnp.maximum(
        jnp.dot(xb, mw1_ref[...].T, preferred_element_type=jnp.float32)
        + mb1_ref[...], 0.0)
    msg = jnp.dot(h, mw2_ref[...].T, preferred_element_type=jnp.float32) \
        + mb2_ref[...]
    g = jnp.maximum(
        jnp.dot(xb, sw1_ref[...].T, preferred_element_type=jnp.float32)
        + sb1_ref[...], 0.0)
    slf = jnp.dot(g, sw2_ref[...].T, preferred_element_type=jnp.float32) \
        + sb2_ref[...]
    m2_ref[0:N] = msg[:, 0:HH]
    m2_ref[N:2 * N] = msg[:, HH:H]
    s2_ref[0:N] = slf[:, 0:HH]
    s2_ref[N:2 * N] = slf[:, HH:H]


_mlp = pl.pallas_call(
    _mlp_body,
    out_shape=(jax.ShapeDtypeStruct((NC * N, HH), jnp.float32),
               jax.ShapeDtypeStruct((NC * N, HH), jnp.float32)),
)


@functools.partial(
    pl.kernel,
    out_type=jax.ShapeDtypeStruct((NC, N, HH), jnp.float32),
    mesh=plsc.VectorSubcoreMesh(core_axis_name="c", subcore_axis_name="s"),
    compiler_params=pltpu.CompilerParams(needs_layout_passes=False,
                                         use_tc_tiling_on_sc=False),
    scratch_types=[
        pltpu.VMEM((IDXR, GRP), jnp.int32),
        pltpu.VMEM((IDXR, GRP), jnp.int32),
        pltpu.VMEM((IDXR, GRP), jnp.int32),
        pltpu.VMEM((IDXR, GRP), jnp.int32),
        pltpu.VMEM((IDXR, GRP), jnp.int32),
        pltpu.VMEM((IDXR, GRP), jnp.int32),
        pltpu.VMEM((4 * SUB,), jnp.float32),
        pltpu.VMEM((4 * SUB,), jnp.float32),
        pltpu.VMEM((4 * SUB,), jnp.float32),
        pltpu.VMEM((SUB, HH), jnp.float32),
        pltpu.VMEM((SUB, HH), jnp.float32),
        pltpu.VMEM((SUB, HH), jnp.float32),
        pltpu.VMEM_SHARED((N, HH), jnp.float32),
        pltpu.SemaphoreType.DMA,
        pltpu.SemaphoreType.DMA,
        pltpu.SemaphoreType.DMA,
        pltpu.SemaphoreType.DMA,
        pltpu.SemaphoreType.DMA,
        pltpu.SemaphoreType.DMA,
        pltpu.SemaphoreType.DMA,
        pltpu.SemaphoreType.DMA,
        pltpu.SemaphoreType.DMA,
    ],
)
def _sc_aggr(m2_hbm, s2_hbm, src_hbm, dst_hbm, w_hbm, out_hbm,
             srcm0, srcm1, srcm2, dstm0, dstm1, dstm2, wv0, wv1, wv2,
             rows0, rows1, rows2, accum,
             isem0, isem1, isem2, gsem0, gsem1, gsem2, ssem0, ssem1, ssem2):
    c = lax.axis_index("c")
    s = lax.axis_index("s")
    srcm = (srcm0, srcm1, srcm2)
    dstm = (dstm0, dstm1, dstm2)
    wv = (wv0, wv1, wv2)
    rows = (rows0, rows1, rows2)
    isem = (isem0, isem1, isem2)
    gsem = (gsem0, gsem1, gsem2)
    ssem = (ssem0, ssem1, ssem2)

    pltpu.sync_copy(s2_hbm.at[pl.ds(c * N + s * SLAB, SLAB)],
                    accum.at[pl.ds(s * SLAB, SLAB)])

    @pl.when(s == NT - 1)
    def _init_tail():
        pltpu.sync_copy(s2_hbm.at[pl.ds(c * N + NT * SLAB, N - NT * SLAB)],
                        accum.at[pl.ds(NT * SLAB, N - NT * SLAB)])
    plsc.subcore_barrier()

    def scale(r, p, qq):
        def body(b4, _):
            for uu in range(4):
                b = b4 * 4 + uu
                wb = plsc.load_gather(
                    wv[p], [jnp.full((16,), qq * SUB + b, jnp.int32)])
                for t in range(HH // 16):
                    sl = pl.ds(t * 16, 16)
                    rows[r][b, sl] = rows[r][b, sl] * wb
            return 0
        lax.fori_loop(0, SUB // 4, body, 0)

    def block(blki, _):
        idescs = []
        for p in range(3):
            qb = s * (EPT // GRP) + (blki * 3 + p) * IDXR
            eb = s * EPT + (blki * 3 + p) * (4 * SUB)
            idescs.append((
                pltpu.async_copy(src_hbm.at[c, pl.ds(qb, IDXR)],
                                 srcm[p], isem[p]),
                pltpu.async_copy(dst_hbm.at[pl.ds(qb, IDXR)],
                                 dstm[p], isem[p]),
                pltpu.async_copy(w_hbm.at[pl.ds(eb, 4 * SUB)],
                                 wv[p], isem[p]),
            ))

        gd = {}
        sd = {}

        def gather_issue(k):
            p, qq, r = k // 4, k % 4, k % 3
            gd[k] = [
                pltpu.async_copy(
                    m2_hbm.at[srcm[p].at[qq * (SUB // GRP) + j]],
                    rows[r].at[pl.ds(j * GRP, GRP)], gsem[r])
                for j in range(SUB // GRP)
            ]

        def scatter_issue(k):
            p, qq, r = k // 4, k % 4, k % 3
            sd[k] = [
                pltpu.async_copy(
                    rows[r].at[pl.ds(j * GRP, GRP)],
                    accum.at[dstm[p].at[qq * (SUB // GRP) + j]],
                    ssem[r], add=True)
                for j in range(SUB // GRP)
            ]

        for d in idescs[0]:
            d.wait()
        gather_issue(0)
        gather_issue(1)

        for u in range(BLK):
            for d in gd[u]:
                d.wait()
            if u + 2 < BLK:
                if (u + 2) % 4 == 0:
                    for d in idescs[(u + 2) // 4]:
                        d.wait()
                gather_issue(u + 2)
        return 0

    lax.fori_loop(0, NBLK, block, 0)

    plsc.subcore_barrier()
    pltpu.sync_copy(accum.at[pl.ds(s * SLAB, SLAB)],
                    out_hbm.at[c, pl.ds(s * SLAB, SLAB)])

    @pl.when(s == NT - 1)
    def _out_tail():
        pltpu.sync_copy(accum.at[pl.ds(NT * SLAB, N - NT * SLAB)],
                        out_hbm.at[c, pl.ds(NT * SLAB, N - NT * SLAB)])


def kernel(x, edge_index, edge_weights, mW1, mb1, mW2, mb2, sW1, sb1, sW2, sb2):
    ei = edge_index.astype(jnp.int32)
    e = ei.shape[1]
    pad = E_PAD - e
    src0 = jnp.concatenate([ei[0], jnp.zeros((pad,), jnp.int32)])
    src = jnp.stack([src0, src0 + N]).reshape(NC, -1, GRP)
    dst = jnp.concatenate([ei[1], jnp.zeros((pad,), jnp.int32)]).reshape(-1, GRP)
    w = jnp.concatenate([edge_weights.astype(jnp.float32),
                         jnp.zeros((pad,), jnp.float32)])
    m2, s2 = _mlp(x, mW1, mb1.reshape(1, H), mW2, mb2.reshape(1, H),
                  sW1, sb1.reshape(1, H), sW2, sb2.reshape(1, H))
    out2 = _sc_aggr(m2, s2, src, dst, w)
    return out2.transpose(1, 0, 2).reshape(N, H)


# speedup vs baseline: 2.9904x; 2.9904x over previous
import functools

import jax
import jax.numpy as jnp
from jax import lax
from jax.experimental import pallas as pl
from jax.experimental.pallas import tpu as pltpu
from jax.experimental.pallas import tpu_sc as plsc

H = 128
HH = H // 2
N = 10000
NC = 2
NT = 16
SLAB = 624
GRP = 128
SUB = GRP
BLK = 24
NBLK = 7
EPT = SUB * BLK * NBLK
E_PAD = EPT * NT
IDXR = 8


def _mlp_body(x_ref, mw1_ref, mb1_ref, mw2_ref, mb2_ref,
              sw1_ref, sb1_ref, sw2_ref, sb2_ref, m2_ref, s2_ref):
    xb = x_ref[...]
    h = jnp.maximum(
        jnp.dot(xb, mw1_ref[...].T, preferred_element_type=jnp.float32)
        + mb1_ref[...], 0.0)
    msg = jnp.dot(h, mw2_ref[...].T, preferred_element_type=jnp.float32) \
        + mb2_ref[...]
    g = jnp.maximum(
        jnp.dot(xb, sw1_ref[...].T, preferred_element_type=jnp.float32)
        + sb1_ref[...], 0.0)
    slf = jnp.dot(g, sw2_ref[...].T, preferred_element_type=jnp.float32) \
        + sb2_ref[...]
    m2_ref[0:N] = msg[:, 0:HH]
    m2_ref[N:2 * N] = msg[:, HH:H]
    s2_ref[0:N] = slf[:, 0:HH]
    s2_ref[N:2 * N] = slf[:, HH:H]


_mlp = pl.pallas_call(
    _mlp_body,
    out_shape=(jax.ShapeDtypeStruct((NC * N, HH), jnp.float32),
               jax.ShapeDtypeStruct((NC * N, HH), jnp.float32)),
)


@functools.partial(
    pl.kernel,
    out_type=jax.ShapeDtypeStruct((NC, N, HH), jnp.float32),
    mesh=plsc.VectorSubcoreMesh(core_axis_name="c", subcore_axis_name="s"),
    compiler_params=pltpu.CompilerParams(needs_layout_passes=False,
                                         use_tc_tiling_on_sc=False),
    scratch_types=[
        pltpu.VMEM((IDXR, GRP), jnp.int32),
        pltpu.VMEM((IDXR, GRP), jnp.int32),
        pltpu.VMEM((IDXR, GRP), jnp.int32),
        pltpu.VMEM((IDXR, GRP), jnp.int32),
        pltpu.VMEM((IDXR, GRP), jnp.int32),
        pltpu.VMEM((IDXR, GRP), jnp.int32),
        pltpu.VMEM((IDXR * GRP,), jnp.float32),
        pltpu.VMEM((IDXR * GRP,), jnp.float32),
        pltpu.VMEM((IDXR * GRP,), jnp.float32),
        pltpu.VMEM((SUB, HH), jnp.float32),
        pltpu.VMEM((SUB, HH), jnp.float32),
        pltpu.VMEM((SUB, HH), jnp.float32),
        pltpu.VMEM_SHARED((N, HH), jnp.float32),
        pltpu.VMEM_SHARED((N, HH), jnp.float32),
        pltpu.SemaphoreType.DMA,
        pltpu.SemaphoreType.DMA,
        pltpu.SemaphoreType.DMA,
        pltpu.SemaphoreType.DMA,
        pltpu.SemaphoreType.DMA,
        pltpu.SemaphoreType.DMA,
        pltpu.SemaphoreType.DMA,
        pltpu.SemaphoreType.DMA,
        pltpu.SemaphoreType.DMA,
    ],
)
def _sc_aggr(m2_hbm, s2_hbm, src_hbm, dst_hbm, w_hbm, out_hbm,
             srcm0, srcm1, srcm2, dstm0, dstm1, dstm2, wv0, wv1, wv2,
             rows0, rows1, rows2, msp, accum,
             isem0, isem1, isem2, gsem0, gsem1, gsem2, ssem0, ssem1, ssem2):
    c = lax.axis_index("c")
    s = lax.axis_index("s")
    srcm = (srcm0, srcm1, srcm2)
    dstm = (dstm0, dstm1, dstm2)
    wv = (wv0, wv1, wv2)
    rows = (rows0, rows1, rows2)
    isem = (isem0, isem1, isem2)
    gsem = (gsem0, gsem1, gsem2)
    ssem = (ssem0, ssem1, ssem2)

    pltpu.sync_copy(m2_hbm.at[pl.ds(c * N + s * SLAB, SLAB)],
                    msp.at[pl.ds(s * SLAB, SLAB)])
    pltpu.sync_copy(s2_hbm.at[pl.ds(c * N + s * SLAB, SLAB)],
                    accum.at[pl.ds(s * SLAB, SLAB)])

    @pl.when(s == NT - 1)
    def _init_tail():
        pltpu.sync_copy(m2_hbm.at[pl.ds(c * N + NT * SLAB, N - NT * SLAB)],
                        msp.at[pl.ds(NT * SLAB, N - NT * SLAB)])
        pltpu.sync_copy(s2_hbm.at[pl.ds(c * N + NT * SLAB, N - NT * SLAB)],
                        accum.at[pl.ds(NT * SLAB, N - NT * SLAB)])
    plsc.subcore_barrier()

    def scale(r, p, qq):
        def body(b4, _):
            for uu in range(4):
                b = b4 * 4 + uu
                wb = plsc.load_gather(
                    wv[p], [jnp.full((16,), qq * SUB + b, jnp.int32)])
                for t in range(HH // 16):
                    sl = pl.ds(t * 16, 16)
                    rows[r][b, sl] = rows[r][b, sl] * wb
            return 0
        lax.fori_loop(0, SUB // 4, body, 0)

    def block(blki, _):
        idescs = []
        for p in range(3):
            qb = s * (EPT // GRP) + (blki * 3 + p) * IDXR
            eb = s * EPT + (blki * 3 + p) * (IDXR * GRP)
            idescs.append((
                pltpu.async_copy(src_hbm.at[pl.ds(qb, IDXR)],
                                 srcm[p], isem[p]),
                pltpu.async_copy(dst_hbm.at[pl.ds(qb, IDXR)],
                                 dstm[p], isem[p]),
                pltpu.async_copy(w_hbm.at[pl.ds(eb, IDXR * GRP)],
                                 wv[p], isem[p]),
            ))

        gd = {}
        sd = {}

        def gather_issue(k):
            p, qq, r = k // IDXR, k % IDXR, k % 3
            gd[k] = pltpu.async_copy(msp.at[srcm[p].at[qq]], rows[r], gsem[r])

        def scatter_issue(k):
            p, qq, r = k // IDXR, k % IDXR, k % 3
            sd[k] = pltpu.async_copy(rows[r], accum.at[dstm[p].at[qq]],
                                     ssem[r], add=True)

        for d in idescs[0]:
            d.wait()
        gather_issue(0)
        gather_issue(1)

        for u in range(BLK):
            gd[u].wait()
            scale(u % 3, u // IDXR, u % IDXR)
            scatter_issue(u)
            if u + 2 < BLK:
                if u >= 1:
                    sd[u - 1].wait()
                if (u + 2) % IDXR == 0:
                    for d in idescs[(u + 2) // IDXR]:
                        d.wait()
                gather_issue(u + 2)

        for u in (BLK - 3, BLK - 2, BLK - 1):
            sd[u].wait()
        return 0

    lax.fori_loop(0, NBLK, block, 0)

    plsc.subcore_barrier()
    pltpu.sync_copy(accum.at[pl.ds(s * SLAB, SLAB)],
                    out_hbm.at[c, pl.ds(s * SLAB, SLAB)])

    @pl.when(s == NT - 1)
    def _out_tail():
        pltpu.sync_copy(accum.at[pl.ds(NT * SLAB, N - NT * SLAB)],
                        out_hbm.at[c, pl.ds(NT * SLAB, N - NT * SLAB)])


def kernel(x, edge_index, edge_weights, mW1, mb1, mW2, mb2, sW1, sb1, sW2, sb2):
    ei = edge_index.astype(jnp.int32)
    e = ei.shape[1]
    pad = E_PAD - e
    src = jnp.concatenate([ei[0], jnp.zeros((pad,), jnp.int32)]).reshape(-1, GRP)
    dst = jnp.concatenate([ei[1], jnp.zeros((pad,), jnp.int32)]).reshape(-1, GRP)
    w = jnp.concatenate([edge_weights.astype(jnp.float32),
                         jnp.zeros((pad,), jnp.float32)])
    m2, s2 = _mlp(x, mW1, mb1.reshape(1, H), mW2, mb2.reshape(1, H),
                  sW1, sb1.reshape(1, H), sW2, sb2.reshape(1, H))
    out2 = _sc_aggr(m2, s2, src, dst, w)
    return out2.transpose(1, 0, 2).reshape(N, H)


# speedup vs baseline: 3.7724x; 1.2615x over previous
import functools

import jax
import jax.numpy as jnp
from jax import lax
from jax.experimental import pallas as pl
from jax.experimental.pallas import tpu as pltpu
from jax.experimental.pallas import tpu_sc as plsc

H = 128
HH = H // 2
N = 10000
NC = 2
NT = 16
SLAB = 624
GRP = 128
SUB = GRP
BLK = 24
NBLK = 7
EPT = SUB * BLK * NBLK
E_PAD = EPT * NT
IDXR = 8


def _mlp_body(x_ref, mw1_ref, mb1_ref, mw2_ref, mb2_ref,
              sw1_ref, sb1_ref, sw2_ref, sb2_ref, m2_ref, s2_ref):
    xb = x_ref[...]
    h = jnp.maximum(
        jnp.dot(xb, mw1_ref[...].T, preferred_element_type=jnp.float32)
        + mb1_ref[...], 0.0)
    msg = jnp.dot(h, mw2_ref[...].T, preferred_element_type=jnp.float32) \
        + mb2_ref[...]
    g = jnp.maximum(
        jnp.dot(xb, sw1_ref[...].T, preferred_element_type=jnp.float32)
        + sb1_ref[...], 0.0)
    slf = jnp.dot(g, sw2_ref[...].T, preferred_element_type=jnp.float32) \
        + sb2_ref[...]
    m2_ref[0:N] = msg[:, 0:HH]
    m2_ref[N:2 * N] = msg[:, HH:H]
    s2_ref[0:N] = slf[:, 0:HH]
    s2_ref[N:2 * N] = slf[:, HH:H]


_mlp = pl.pallas_call(
    _mlp_body,
    out_shape=(jax.ShapeDtypeStruct((NC * N, HH), jnp.float32),
               jax.ShapeDtypeStruct((NC * N, HH), jnp.float32)),
)


@functools.partial(
    pl.kernel,
    out_type=jax.ShapeDtypeStruct((NC, N, HH), jnp.float32),
    mesh=plsc.VectorSubcoreMesh(core_axis_name="c", subcore_axis_name="s"),
    compiler_params=pltpu.CompilerParams(needs_layout_passes=False,
                                         use_tc_tiling_on_sc=False),
    scratch_types=[
        pltpu.VMEM((IDXR, GRP), jnp.int32),
        pltpu.VMEM((IDXR, GRP), jnp.int32),
        pltpu.VMEM((IDXR, GRP), jnp.int32),
        pltpu.VMEM((IDXR, GRP), jnp.int32),
        pltpu.VMEM((IDXR, GRP), jnp.int32),
        pltpu.VMEM((IDXR, GRP), jnp.int32),
        pltpu.VMEM((IDXR * GRP,), jnp.float32),
        pltpu.VMEM((IDXR * GRP,), jnp.float32),
        pltpu.VMEM((IDXR * GRP,), jnp.float32),
        pltpu.VMEM((SUB, HH), jnp.float32),
        pltpu.VMEM((SUB, HH), jnp.float32),
        pltpu.VMEM((SUB, HH), jnp.float32),
        pltpu.VMEM_SHARED((N, HH), jnp.float32),
        pltpu.VMEM_SHARED((N, HH), jnp.float32),
        pltpu.SemaphoreType.DMA,
        pltpu.SemaphoreType.DMA,
        pltpu.SemaphoreType.DMA,
        pltpu.SemaphoreType.DMA,
        pltpu.SemaphoreType.DMA,
        pltpu.SemaphoreType.DMA,
        pltpu.SemaphoreType.DMA,
        pltpu.SemaphoreType.DMA,
        pltpu.SemaphoreType.DMA,
    ],
)
def _sc_aggr(m2_hbm, s2_hbm, src_hbm, dst_hbm, w_hbm, out_hbm,
             srcm0, srcm1, srcm2, dstm0, dstm1, dstm2, wv0, wv1, wv2,
             rows0, rows1, rows2, msp, accum,
             isem0, isem1, isem2, gsem0, gsem1, gsem2, ssem0, ssem1, ssem2):
    c = lax.axis_index("c")
    s = lax.axis_index("s")
    srcm = (srcm0, srcm1, srcm2)
    dstm = (dstm0, dstm1, dstm2)
    wv = (wv0, wv1, wv2)
    rows = (rows0, rows1, rows2)
    isem = (isem0, isem1, isem2)
    gsem = (gsem0, gsem1, gsem2)
    ssem = (ssem0, ssem1, ssem2)

    pltpu.sync_copy(m2_hbm.at[pl.ds(c * N + s * SLAB, SLAB)],
                    msp.at[pl.ds(s * SLAB, SLAB)])
    pltpu.sync_copy(s2_hbm.at[pl.ds(c * N + s * SLAB, SLAB)],
                    accum.at[pl.ds(s * SLAB, SLAB)])

    @pl.when(s == NT - 1)
    def _init_tail():
        pltpu.sync_copy(m2_hbm.at[pl.ds(c * N + NT * SLAB, N - NT * SLAB)],
                        msp.at[pl.ds(NT * SLAB, N - NT * SLAB)])
        pltpu.sync_copy(s2_hbm.at[pl.ds(c * N + NT * SLAB, N - NT * SLAB)],
                        accum.at[pl.ds(NT * SLAB, N - NT * SLAB)])
    plsc.subcore_barrier()

    def scale(r, p, qq):
        def body(b4, _):
            for uu in range(4):
                b = b4 * 4 + uu
                wb = plsc.load_gather(
                    wv[p], [jnp.full((16,), qq * SUB + b, jnp.int32)])
                for t in range(HH // 16):
                    sl = pl.ds(t * 16, 16)
                    rows[r][b, sl] = rows[r][b, sl] * wb
            return 0
        lax.fori_loop(0, SUB // 4, body, 0)

    def block(blki, _):
        idescs = []
        for p in range(3):
            qb = s * (EPT // GRP) + (blki * 3 + p) * IDXR
            eb = s * EPT + (blki * 3 + p) * (IDXR * GRP)
            idescs.append((
                pltpu.async_copy(src_hbm.at[pl.ds(qb, IDXR)],
                                 srcm[p], isem[p]),
                pltpu.async_copy(dst_hbm.at[pl.ds(qb, IDXR)],
                                 dstm[p], isem[p]),
                pltpu.async_copy(w_hbm.at[pl.ds(eb, IDXR * GRP)],
                                 wv[p], isem[p]),
            ))

        gd = {}
        sd = {}

        def gather_issue(k):
            p, qq, r = k // IDXR, k % IDXR, k % 3
            gd[k] = pltpu.async_copy(msp.at[srcm[p].at[qq]], rows[r], gsem[r])

        def scatter_issue(k):
            p, qq, r = k // IDXR, k % IDXR, k % 3
            sd[k] = pltpu.async_copy(rows[r], accum.at[dstm[p].at[qq]],
                                     ssem[r], add=True)

        for d in idescs[0]:
            d.wait()
        gather_issue(0)
        gather_issue(1)

        for u in range(BLK):
            gd[u].wait()
            scatter_issue(u)
            if u + 2 < BLK:
                if u >= 1:
                    sd[u - 1].wait()
                if (u + 2) % IDXR == 0:
                    for d in idescs[(u + 2) // IDXR]:
                        d.wait()
                gather_issue(u + 2)

        for u in (BLK - 3, BLK - 2, BLK - 1):
            sd[u].wait()
        return 0

    lax.fori_loop(0, NBLK, block, 0)

    plsc.subcore_barrier()
    pltpu.sync_copy(accum.at[pl.ds(s * SLAB, SLAB)],
                    out_hbm.at[c, pl.ds(s * SLAB, SLAB)])

    @pl.when(s == NT - 1)
    def _out_tail():
        pltpu.sync_copy(accum.at[pl.ds(NT * SLAB, N - NT * SLAB)],
                        out_hbm.at[c, pl.ds(NT * SLAB, N - NT * SLAB)])


def kernel(x, edge_index, edge_weights, mW1, mb1, mW2, mb2, sW1, sb1, sW2, sb2):
    ei = edge_index.astype(jnp.int32)
    e = ei.shape[1]
    pad = E_PAD - e
    src = jnp.concatenate([ei[0], jnp.zeros((pad,), jnp.int32)]).reshape(-1, GRP)
    dst = jnp.concatenate([ei[1], jnp.zeros((pad,), jnp.int32)]).reshape(-1, GRP)
    w = jnp.concatenate([edge_weights.astype(jnp.float32),
                         jnp.zeros((pad,), jnp.float32)])
    m2, s2 = _mlp(x, mW1, mb1.reshape(1, H), mW2, mb2.reshape(1, H),
                  sW1, sb1.reshape(1, H), sW2, sb2.reshape(1, H))
    out2 = _sc_aggr(m2, s2, src, dst, w)
    return out2.transpose(1, 0, 2).reshape(N, H)
